# baseline (speedup 1.0000x reference)
import jax
import jax.numpy as jnp
from jax import lax
from jax.experimental import pallas as pl
from jax.experimental.pallas import tpu as pltpu

_HEADS = 12
_DH = 64
_N = 4096
_H2 = _N // 2
_D = 768
_LOGN = 12
_TILE = 512
_NT2 = _H2 // _TILE


def _mm_t(a, b):
    return lax.dot_general(a, b, (((1,), (1,)), ((), ())),
                           preferred_element_type=jnp.float32)


def _mm(a, b):
    return lax.dot_general(a, b, (((1,), (0,)), ((), ())),
                           preferred_element_type=jnp.float32)


def _v_kernel(x_ref, wv_ref, o_ref):
    o_ref[...] = _mm_t(x_ref[...].astype(jnp.bfloat16), wv_ref[...])


def _stage_kernel(v_ref, wqk_ref, out_ref, ha_ref, hb_ref, vs_ref,
                  ra_ref, rb_ref):
    s = pl.program_id(0)

    @pl.when(s == 0)
    def _():
        out_ref[...] = v_ref[...].astype(jnp.bfloat16)

    for c in range(_LOGN):
        @pl.when(s == c)
        def _(c=c):
            st = 1 << c
            g = _N // (2 * st)
            h4 = out_ref[...].reshape(g, 2, st, _D)
            ha_ref[...] = h4[:, 0].reshape(_H2, _D)
            hb_ref[...] = h4[:, 1].reshape(_H2, _D)
            v4 = v_ref[...].reshape(g, 2, st, _D)
            vs_ref[...] = (v4[:, 0] + v4[:, 1]).reshape(_H2, _D)

    wq = wqk_ref[0, :_D, :]
    wk = wqk_ref[0, _D:, :]

    scale = jnp.float32(_DH ** -0.5)
    seg = (lax.broadcasted_iota(jnp.int32, (_D, _HEADS), 0) // _DH ==
           lax.broadcasted_iota(jnp.int32, (_D, _HEADS), 1))
    smat = jnp.where(seg, scale, jnp.float32(0.0)).astype(jnp.bfloat16)

    rep = (lax.broadcasted_iota(jnp.int32, (_HEADS, _D), 0) ==
           lax.broadcasted_iota(jnp.int32, (_HEADS, _D), 1) // _DH)
    bmat = jnp.where(rep, jnp.float32(1.0), jnp.float32(0.0)).astype(jnp.bfloat16)

    for t in range(_NT2):
        rows = pl.ds(t * _TILE, _TILE)
        ha_t = ha_ref[rows, :]
        hb_t = hb_ref[rows, :]
        qa = _mm_t(ha_t, wq)
        ka = _mm_t(ha_t, wk)
        kb = _mm_t(hb_t, wk)
        e0 = _mm((qa * ka).astype(jnp.bfloat16), smat)
        e1 = _mm((qa * kb).astype(jnp.bfloat16), smat)
        w0 = jax.nn.sigmoid(e0 - e1)
        w1 = 1.0 - w0
        wf0 = _mm(w0.astype(jnp.bfloat16), bmat)
        wf1 = _mm(w1.astype(jnp.bfloat16), bmat)
        vs_t = vs_ref[rows, :]
        ra_ref[rows, :] = (wf0 * vs_t).astype(jnp.bfloat16)
        rb_ref[rows, :] = (wf1 * vs_t).astype(jnp.bfloat16)

    for c in range(_LOGN):
        @pl.when(s == c)
        def _(c=c):
            st = 1 << c
            g = _N // (2 * st)
            ra4 = ra_ref[...].reshape(g, 1, st, _D)
            rb4 = rb_ref[...].reshape(g, 1, st, _D)
            out_ref[...] = jnp.concatenate([ra4, rb4], axis=1).reshape(_N, _D)


def _run(x2, W_v, W_qk, interpret=False):
    v = pl.pallas_call(
        _v_kernel,
        grid=(_N // 512,),
        in_specs=[pl.BlockSpec((512, _D), lambda i: (i, 0)),
                  pl.BlockSpec((_D, _D), lambda i: (0, 0))],
        out_specs=pl.BlockSpec((512, _D), lambda i: (i, 0)),
        out_shape=jax.ShapeDtypeStruct((_N, _D), jnp.float32),
        interpret=interpret,
    )(x2, W_v.astype(jnp.bfloat16))

    h = pl.pallas_call(
        _stage_kernel,
        grid=(_LOGN,),
        in_specs=[pl.BlockSpec((_N, _D), lambda s: (0, 0)),
                  pl.BlockSpec((1, 2 * _D, _D), lambda s: (s, 0, 0))],
        out_specs=pl.BlockSpec((_N, _D), lambda s: (0, 0)),
        out_shape=jax.ShapeDtypeStruct((_N, _D), jnp.bfloat16),
        scratch_shapes=[pltpu.VMEM((_H2, _D), jnp.bfloat16),
                        pltpu.VMEM((_H2, _D), jnp.bfloat16),
                        pltpu.VMEM((_H2, _D), jnp.float32),
                        pltpu.VMEM((_H2, _D), jnp.bfloat16),
                        pltpu.VMEM((_H2, _D), jnp.bfloat16)],
        interpret=interpret,
    )(v, W_qk.astype(jnp.bfloat16))
    return h


def kernel(x, W_v, W_qk):
    B, N, D = x.shape
    h = _run(x.reshape(N, D), W_v, W_qk)
    return h.astype(jnp.float32).reshape(B, N, D)
